# Initial kernel scaffold; baseline (speedup 1.0000x reference)
import jax
import jax.numpy as jnp
from jax.experimental import pallas as pl


def kernel(x, eps, enc_rnn1, enc_mean, enc_var, dec_rnn1, dec_rnn2, dec_rnn3):
    raise NotImplementedError("write your pallas kernel here")


# speedup vs baseline: 5.3879x; 5.3879x over previous
import jax
import jax.numpy as jnp
from jax.experimental import pallas as pl
from jax.experimental.pallas import tpu as pltpu

_T = 512
_B = 64
_BB = 32
_FLAT = 512


def _prep_dir(p):
    Wih, Whh, bih, bhh = p
    return jnp.transpose(Wih), jnp.transpose(Whh), (bih + bhh)[None, :]


def _cell(g, c, H):
    i = jax.nn.sigmoid(g[:, 0 * H:1 * H])
    f = jax.nn.sigmoid(g[:, 1 * H:2 * H])
    gg = jnp.tanh(g[:, 2 * H:3 * H])
    o = jax.nn.sigmoid(g[:, 3 * H:4 * H])
    c2 = f * c + i * gg
    h2 = o * jnp.tanh(c2)
    return h2, c2


def _fused_layer(runs):
    init = tuple(
        (jnp.zeros((_BB, r["H"]), jnp.float32),
         jnp.zeros((_BB, r["H"]), jnp.float32))
        for r in runs
    )

    def body(t, carry):
        out = []
        for r, (h, c) in zip(runs, carry):
            te = _T - 1 - t if r["rev"] else t
            g = r["gx"](te) + jnp.dot(h, r["WhhT"],
                                      preferred_element_type=jnp.float32)
            h2, c2 = _cell(g, c, r["H"])
            if r["write"] is not None:
                r["write"](te, h2)
            out.append((h2, c2))
        return tuple(out)

    return jax.lax.fori_loop(0, _T, body, init)


def _mk_read(buf, C):
    def rd(t):
        return buf[pl.ds(t, 1)].reshape(_BB, 128)[:, :C]
    return rd


def _mk_write(buf, H, off):
    def wr(t, h):
        buf[pl.ds(t, 1), :, off:off + H] = h.reshape(1, _BB, H)
    return wr


def _std_run(rd, WihT, b, WhhT, H, write, rev):
    def gx(t):
        return jnp.dot(rd(t), WihT, preferred_element_type=jnp.float32) + b
    return dict(gx=gx, WhhT=WhhT, H=H, write=write, rev=rev)


def _body(x_ref, eps_ref, *refs):
    nw = 87
    w = [r[...] for r in refs[:nw]]
    d_ref, mu_ref, lv_ref = refs[nw:nw + 3]
    bufA, bufB, bufC, bufD = refs[nw + 3:nw + 7]

    cur = [0]

    def take3():
        i = cur[0]
        cur[0] = i + 3
        return w[i], w[i + 1], w[i + 2]

    def gx_l0(WihT, b):
        def gx(t):
            xt = x_ref[0, pl.ds(t, 1), :]
            og = jax.lax.dot_general(
                xt, WihT, (((0,), (0,)), ((), ())),
                preferred_element_type=jnp.float32)
            return og + b
        return gx

    enc1_bufs = [bufA, bufB, bufA, bufB]
    src = None
    for l in range(4):
        out = enc1_bufs[l]
        runs = []
        for d in range(2):
            WihT, WhhT, b = take3()
            wr = _mk_write(out, 32, 32 * d)
            if l == 0:
                runs.append(dict(gx=gx_l0(WihT, b), WhhT=WhhT, H=32,
                                 write=wr, rev=(d == 1)))
            else:
                runs.append(_std_run(_mk_read(src, 64), WihT, b, WhhT, 32,
                                     wr, d == 1))
        _fused_layer(runs)
        src = out

    mean_w = [take3() for _ in range(8)]
    var_w = [take3() for _ in range(8)]
    mean_io = [(bufB, 64, bufC), (bufC, 128, bufA),
               (bufA, 128, bufC), (bufC, 128, None)]
    var_io = [(bufB, 64, bufD), (bufD, 128, bufB),
              (bufB, 128, bufD), (bufD, 128, None)]
    for l in range(4):
        runs = []
        for path, io, ws in ((0, mean_io[l], mean_w[2 * l:2 * l + 2]),
                             (1, var_io[l], var_w[2 * l:2 * l + 2])):
            src, C, out = io
            for d in range(2):
                WihT, WhhT, b = ws[d]
                wr = None if out is None else _mk_write(out, 64, 64 * d)
                runs.append(_std_run(_mk_read(src, C), WihT, b, WhhT, 64,
                                     wr, d == 1))
        finals = _fused_layer(runs)
        sl_f = slice(l * 128, l * 128 + 64)
        sl_b = slice(l * 128 + 64, l * 128 + 128)
        mu_ref[:, sl_f] = finals[0][0]
        mu_ref[:, sl_b] = finals[1][0]
        lv_ref[:, sl_f] = finals[2][0]
        lv_ref[:, sl_b] = finals[3][0]

    mu_v = mu_ref[...]
    lv_v = lv_ref[...]
    z = mu_v + jnp.exp(0.5 * lv_v) * eps_ref[...]

    runs = []
    for d in range(2):
        WihT, WhhT, b = take3()
        gz = jnp.dot(z, WihT, preferred_element_type=jnp.float32) + b
        runs.append(dict(gx=(lambda g: (lambda t: g))(gz), WhhT=WhhT, H=32,
                         write=_mk_write(bufA, 32, 32 * d), rev=(d == 1)))
    _fused_layer(runs)

    runs = []
    for d in range(2):
        WihT, WhhT, b = take3()
        runs.append(_std_run(_mk_read(bufA, 64), WihT, b, WhhT, 64,
                             _mk_write(bufB, 64, 64 * d), d == 1))
    _fused_layer(runs)

    Wih3, whh3, b3 = take3()

    def body3(t, carry):
        h, c = carry
        xt = bufB[pl.ds(t, 1)].reshape(_BB, 128)
        g = jax.lax.dot_general(
            Wih3, xt, (((1,), (1,)), ((), ())),
            preferred_element_type=jnp.float32)
        g = g + whh3 * h + b3
        i = jax.nn.sigmoid(g[0:1, :])
        f = jax.nn.sigmoid(g[1:2, :])
        gg = jnp.tanh(g[2:3, :])
        o = jax.nn.sigmoid(g[3:4, :])
        c2 = f * c + i * gg
        h2 = o * jnp.tanh(c2)
        d_ref[0, pl.ds(t, 1), :] = h2
        return (h2, c2)

    jax.lax.fori_loop(0, _T, body3,
                      (jnp.zeros((1, _BB), jnp.float32),
                       jnp.zeros((1, _BB), jnp.float32)))


def kernel(x, eps, enc_rnn1, enc_mean, enc_var, dec_rnn1, dec_rnn2, dec_rnn3):
    x2 = jnp.transpose(x[:, :, 0].reshape(2, _BB, _T), (0, 2, 1))

    weights = []
    for module in (enc_rnn1, enc_mean, enc_var, dec_rnn1, dec_rnn2):
        for layer in module:
            for p in layer:
                weights.extend(_prep_dir(p))
    Wih3, Whh3, bih3, bhh3 = dec_rnn3[0][0]
    weights.extend([Wih3, Whh3, (bih3 + bhh3)[:, None]])

    def wspec(a):
        nd = a.ndim
        return pl.BlockSpec(a.shape, (lambda i, _nd=nd: (0,) * _nd))

    in_specs = [
        pl.BlockSpec((1, _T, _BB), lambda i: (i, 0, 0)),
        pl.BlockSpec((_BB, _FLAT), lambda i: (i, 0)),
    ] + [wspec(a) for a in weights]

    out_shape = [
        jax.ShapeDtypeStruct((2, _T, _BB), jnp.float32),
        jax.ShapeDtypeStruct((_B, _FLAT), jnp.float32),
        jax.ShapeDtypeStruct((_B, _FLAT), jnp.float32),
    ]
    out_specs = [
        pl.BlockSpec((1, _T, _BB), lambda i: (i, 0, 0)),
        pl.BlockSpec((_BB, _FLAT), lambda i: (i, 0)),
        pl.BlockSpec((_BB, _FLAT), lambda i: (i, 0)),
    ]

    d_tb, mu, log_var = pl.pallas_call(
        _body,
        grid=(2,),
        in_specs=in_specs,
        out_specs=out_specs,
        out_shape=out_shape,
        scratch_shapes=[pltpu.VMEM((_T, _BB, 128), jnp.float32)
                        for _ in range(4)],
        compiler_params=pltpu.CompilerParams(
            dimension_semantics=("arbitrary",),
            vmem_limit_bytes=100 * 1024 * 1024,
        ),
        name="recurrent_autoencoder",
    )(x2, eps, *weights)

    d = jnp.transpose(d_tb, (0, 2, 1)).reshape(_B, _T)[:, :, None]
    return d, mu, log_var


# speedup vs baseline: 6.6749x; 1.2389x over previous
import jax
import jax.numpy as jnp
from jax.experimental import pallas as pl
from jax.experimental.pallas import tpu as pltpu

_T = 512
_B = 64
_FLAT = 512


def _prep_dir(p):
    Wih, Whh, bih, bhh = p
    return jnp.transpose(Wih), jnp.transpose(Whh), (bih + bhh)[None, :]


def _cell(g, c, H):
    i = jax.nn.sigmoid(g[:, 0 * H:1 * H])
    f = jax.nn.sigmoid(g[:, 1 * H:2 * H])
    gg = jnp.tanh(g[:, 2 * H:3 * H])
    o = jax.nn.sigmoid(g[:, 3 * H:4 * H])
    c2 = f * c + i * gg
    h2 = o * jnp.tanh(c2)
    return h2, c2


def _fused_layer(runs):
    init = tuple(
        (jnp.zeros((_B, r["H"]), jnp.float32),
         jnp.zeros((_B, r["H"]), jnp.float32))
        for r in runs
    )

    def body(t, carry):
        out = []
        for r, (h, c) in zip(runs, carry):
            te = _T - 1 - t if r["rev"] else t
            g = r["gx"](te) + jnp.dot(h, r["WhhT"],
                                      preferred_element_type=jnp.float32)
            h2, c2 = _cell(g, c, r["H"])
            if r["write"] is not None:
                r["write"](te, h2)
            out.append((h2, c2))
        return tuple(out)

    return jax.lax.fori_loop(0, _T, body, init)


def _mk_read(buf, off, C):
    def rd(t):
        return buf[pl.ds(t, 1)].reshape(_B, 128)[:, off:off + C]
    return rd


def _mk_write(buf, H, off):
    def wr(t, h):
        buf[pl.ds(t, 1), :, off:off + H] = h.reshape(1, _B, H)
    return wr


def _std_run(rd, WihT, b, WhhT, H, write, rev):
    def gx(t):
        return jnp.dot(rd(t), WihT, preferred_element_type=jnp.float32) + b
    return dict(gx=gx, WhhT=WhhT, H=H, write=write, rev=rev)


def _body(x_ref, eps_ref, *refs):
    nw = 87
    w = [r[...] for r in refs[:nw]]
    d_ref, mu_ref, lv_ref = refs[nw:nw + 3]
    bufA, bufB, bufC = refs[nw + 3:nw + 6]

    cur = [0]

    def take3():
        i = cur[0]
        cur[0] = i + 3
        return w[i], w[i + 1], w[i + 2]

    def gx_l0(WihT, b):
        def gx(t):
            xt = x_ref[pl.ds(t, 1), :]
            og = jax.lax.dot_general(
                xt, WihT, (((0,), (0,)), ((), ())),
                preferred_element_type=jnp.float32)
            return og + b
        return gx

    for l in range(4):
        out_off = 64 * (l % 2)
        in_off = 64 * ((l + 1) % 2)
        runs = []
        for d in range(2):
            WihT, WhhT, b = take3()
            wr = _mk_write(bufA, 32, out_off + 32 * d)
            if l == 0:
                runs.append(dict(gx=gx_l0(WihT, b), WhhT=WhhT, H=32,
                                 write=wr, rev=(d == 1)))
            else:
                runs.append(_std_run(_mk_read(bufA, in_off, 64), WihT, b,
                                     WhhT, 32, wr, d == 1))
        _fused_layer(runs)

    mean_w = [take3() for _ in range(8)]
    var_w = [take3() for _ in range(8)]
    path_io = [((bufA, 64, 64), bufB), ((bufB, 0, 128), bufC),
               ((bufC, 0, 128), bufB), ((bufB, 0, 128), None)]
    var_io = [((bufA, 64, 64), bufC), ((bufC, 0, 128), bufB),
              ((bufB, 0, 128), bufC), ((bufC, 0, 128), None)]
    for ws_all, io_all, out_ref in ((mean_w, path_io, mu_ref),
                                    (var_w, var_io, lv_ref)):
        for l in range(4):
            (src, soff, C), out = io_all[l]
            runs = []
            for d in range(2):
                WihT, WhhT, b = ws_all[2 * l + d]
                wr = None if out is None else _mk_write(out, 64, 64 * d)
                runs.append(_std_run(_mk_read(src, soff, C), WihT, b,
                                     WhhT, 64, wr, d == 1))
            finals = _fused_layer(runs)
            out_ref[:, l * 128:l * 128 + 64] = finals[0][0]
            out_ref[:, l * 128 + 64:l * 128 + 128] = finals[1][0]

    z = mu_ref[...] + jnp.exp(0.5 * lv_ref[...]) * eps_ref[...]

    runs = []
    for d in range(2):
        WihT, WhhT, b = take3()
        gz = jnp.dot(z, WihT, preferred_element_type=jnp.float32) + b
        runs.append(dict(gx=(lambda g: (lambda t: g))(gz), WhhT=WhhT, H=32,
                         write=_mk_write(bufA, 32, 32 * d), rev=(d == 1)))
    _fused_layer(runs)

    runs = []
    for d in range(2):
        WihT, WhhT, b = take3()
        runs.append(_std_run(_mk_read(bufA, 0, 64), WihT, b, WhhT, 64,
                             _mk_write(bufB, 64, 64 * d), d == 1))
    _fused_layer(runs)

    Wih3, whh3, b3 = take3()

    def body3(t, carry):
        h, c = carry
        xt = bufB[pl.ds(t, 1)].reshape(_B, 128)
        g = jax.lax.dot_general(
            Wih3, xt, (((1,), (1,)), ((), ())),
            preferred_element_type=jnp.float32)
        g = g + whh3 * h + b3
        i = jax.nn.sigmoid(g[0:1, :])
        f = jax.nn.sigmoid(g[1:2, :])
        gg = jnp.tanh(g[2:3, :])
        o = jax.nn.sigmoid(g[3:4, :])
        c2 = f * c + i * gg
        h2 = o * jnp.tanh(c2)
        d_ref[pl.ds(t, 1), :] = h2
        return (h2, c2)

    jax.lax.fori_loop(0, _T, body3,
                      (jnp.zeros((1, _B), jnp.float32),
                       jnp.zeros((1, _B), jnp.float32)))


def kernel(x, eps, enc_rnn1, enc_mean, enc_var, dec_rnn1, dec_rnn2, dec_rnn3):
    x2 = jnp.transpose(x[:, :, 0])

    weights = []
    for module in (enc_rnn1, enc_mean, enc_var, dec_rnn1, dec_rnn2):
        for layer in module:
            for p in layer:
                weights.extend(_prep_dir(p))
    Wih3, Whh3, bih3, bhh3 = dec_rnn3[0][0]
    weights.extend([Wih3, Whh3, (bih3 + bhh3)[:, None]])

    out_shape = [
        jax.ShapeDtypeStruct((_T, _B), jnp.float32),
        jax.ShapeDtypeStruct((_B, _FLAT), jnp.float32),
        jax.ShapeDtypeStruct((_B, _FLAT), jnp.float32),
    ]

    d_tb, mu, log_var = pl.pallas_call(
        _body,
        out_shape=out_shape,
        scratch_shapes=[pltpu.VMEM((_T, _B, 128), jnp.float32)
                        for _ in range(3)],
        compiler_params=pltpu.CompilerParams(
            vmem_limit_bytes=100 * 1024 * 1024,
        ),
        name="recurrent_autoencoder",
    )(x2, eps, *weights)

    d = jnp.transpose(d_tb)[:, :, None]
    return d, mu, log_var


# speedup vs baseline: 7.2706x; 1.0892x over previous
import jax
import jax.numpy as jnp
from jax.experimental import pallas as pl
from jax.experimental.pallas import tpu as pltpu

_T = 512
_B = 64
_FLAT = 512


def _prep_dir(p):
    Wih, Whh, bih, bhh = p
    return jnp.transpose(Wih), jnp.transpose(Whh), (bih + bhh)[None, :]


def _cell(g, c, H):
    i = jax.nn.sigmoid(g[:, 0 * H:1 * H])
    f = jax.nn.sigmoid(g[:, 1 * H:2 * H])
    gg = jnp.tanh(g[:, 2 * H:3 * H])
    o = jax.nn.sigmoid(g[:, 3 * H:4 * H])
    c2 = f * c + i * gg
    h2 = o * jnp.tanh(c2)
    return h2, c2


def _fused_layer(runs):
    init = tuple(
        (jnp.zeros((_B, r["H"]), jnp.float32),
         jnp.zeros((_B, r["H"]), jnp.float32))
        for r in runs
    )

    def body(t, carry):
        out = []
        for r, (h, c) in zip(runs, carry):
            te = _T - 1 - t if r["rev"] else t
            g = r["gx"](te) + jnp.dot(h, r["WhhT"],
                                      preferred_element_type=jnp.float32)
            h2, c2 = _cell(g, c, r["H"])
            if r["write"] is not None:
                r["write"](te, h2)
            out.append((h2, c2))
        return tuple(out)

    return jax.lax.fori_loop(0, _T, body, init, unroll=2)


def _mk_read(buf, off, C):
    def rd(t):
        return buf[pl.ds(t, 1)].reshape(_B, 128)[:, off:off + C]
    return rd


def _mk_write(buf, H, off):
    def wr(t, h):
        buf[pl.ds(t, 1), :, off:off + H] = h.reshape(1, _B, H)
    return wr


def _std_run(rd, WihT, b, WhhT, H, write, rev):
    def gx(t):
        return jnp.dot(rd(t), WihT, preferred_element_type=jnp.float32) + b
    return dict(gx=gx, WhhT=WhhT, H=H, write=write, rev=rev)


def _body(x_ref, eps_ref, *refs):
    nw = 87
    w = [r[...] for r in refs[:nw]]
    d_ref, mu_ref, lv_ref = refs[nw:nw + 3]
    bufA, bufB, bufC = refs[nw + 3:nw + 6]

    cur = [0]

    def take3():
        i = cur[0]
        cur[0] = i + 3
        return w[i], w[i + 1], w[i + 2]

    def gx_l0(WihT, b):
        def gx(t):
            xt = x_ref[pl.ds(t, 1), :]
            og = jax.lax.dot_general(
                xt, WihT, (((0,), (0,)), ((), ())),
                preferred_element_type=jnp.float32)
            return og + b
        return gx

    for l in range(4):
        out_off = 64 * (l % 2)
        in_off = 64 * ((l + 1) % 2)
        runs = []
        for d in range(2):
            WihT, WhhT, b = take3()
            wr = _mk_write(bufA, 32, out_off + 32 * d)
            if l == 0:
                runs.append(dict(gx=gx_l0(WihT, b), WhhT=WhhT, H=32,
                                 write=wr, rev=(d == 1)))
            else:
                runs.append(_std_run(_mk_read(bufA, in_off, 64), WihT, b,
                                     WhhT, 32, wr, d == 1))
        _fused_layer(runs)

    mean_w = [take3() for _ in range(8)]
    var_w = [take3() for _ in range(8)]
    path_io = [((bufA, 64, 64), bufB), ((bufB, 0, 128), bufC),
               ((bufC, 0, 128), bufB), ((bufB, 0, 128), None)]
    var_io = [((bufA, 64, 64), bufC), ((bufC, 0, 128), bufB),
              ((bufB, 0, 128), bufC), ((bufC, 0, 128), None)]
    for ws_all, io_all, out_ref in ((mean_w, path_io, mu_ref),
                                    (var_w, var_io, lv_ref)):
        for l in range(4):
            (src, soff, C), out = io_all[l]
            runs = []
            for d in range(2):
                WihT, WhhT, b = ws_all[2 * l + d]
                wr = None if out is None else _mk_write(out, 64, 64 * d)
                runs.append(_std_run(_mk_read(src, soff, C), WihT, b,
                                     WhhT, 64, wr, d == 1))
            finals = _fused_layer(runs)
            out_ref[:, l * 128:l * 128 + 64] = finals[0][0]
            out_ref[:, l * 128 + 64:l * 128 + 128] = finals[1][0]

    z = mu_ref[...] + jnp.exp(0.5 * lv_ref[...]) * eps_ref[...]

    runs = []
    for d in range(2):
        WihT, WhhT, b = take3()
        gz = jnp.dot(z, WihT, preferred_element_type=jnp.float32) + b
        runs.append(dict(gx=(lambda g: (lambda t: g))(gz), WhhT=WhhT, H=32,
                         write=_mk_write(bufA, 32, 32 * d), rev=(d == 1)))
    _fused_layer(runs)

    runs = []
    for d in range(2):
        WihT, WhhT, b = take3()
        runs.append(_std_run(_mk_read(bufA, 0, 64), WihT, b, WhhT, 64,
                             _mk_write(bufB, 64, 64 * d), d == 1))
    _fused_layer(runs)

    Wih3, whh3, b3 = take3()

    def body3(t, carry):
        h, c = carry
        xt = bufB[pl.ds(t, 1)].reshape(_B, 128)
        g = jax.lax.dot_general(
            Wih3, xt, (((1,), (1,)), ((), ())),
            preferred_element_type=jnp.float32)
        g = g + whh3 * h + b3
        i = jax.nn.sigmoid(g[0:1, :])
        f = jax.nn.sigmoid(g[1:2, :])
        gg = jnp.tanh(g[2:3, :])
        o = jax.nn.sigmoid(g[3:4, :])
        c2 = f * c + i * gg
        h2 = o * jnp.tanh(c2)
        d_ref[pl.ds(t, 1), :] = h2
        return (h2, c2)

    jax.lax.fori_loop(0, _T, body3,
                      (jnp.zeros((1, _B), jnp.float32),
                       jnp.zeros((1, _B), jnp.float32)), unroll=2)


def kernel(x, eps, enc_rnn1, enc_mean, enc_var, dec_rnn1, dec_rnn2, dec_rnn3):
    x2 = jnp.transpose(x[:, :, 0])

    weights = []
    for module in (enc_rnn1, enc_mean, enc_var, dec_rnn1, dec_rnn2):
        for layer in module:
            for p in layer:
                weights.extend(_prep_dir(p))
    Wih3, Whh3, bih3, bhh3 = dec_rnn3[0][0]
    weights.extend([Wih3, Whh3, (bih3 + bhh3)[:, None]])

    out_shape = [
        jax.ShapeDtypeStruct((_T, _B), jnp.float32),
        jax.ShapeDtypeStruct((_B, _FLAT), jnp.float32),
        jax.ShapeDtypeStruct((_B, _FLAT), jnp.float32),
    ]

    d_tb, mu, log_var = pl.pallas_call(
        _body,
        out_shape=out_shape,
        scratch_shapes=[pltpu.VMEM((_T, _B, 128), jnp.float32)
                        for _ in range(3)],
        compiler_params=pltpu.CompilerParams(
            vmem_limit_bytes=100 * 1024 * 1024,
        ),
        name="recurrent_autoencoder",
    )(x2, eps, *weights)

    d = jnp.transpose(d_tb)[:, :, None]
    return d, mu, log_var


# speedup vs baseline: 7.6725x; 1.0553x over previous
import jax
import jax.numpy as jnp
from jax.experimental import pallas as pl
from jax.experimental.pallas import tpu as pltpu

_T = 512
_B = 64
_FLAT = 512


def _prep_dir(p):
    Wih, Whh, bih, bhh = p
    return jnp.transpose(Wih), jnp.transpose(Whh), (bih + bhh)[None, :]


def _cell(g, c, H):
    i = jax.nn.sigmoid(g[:, 0 * H:1 * H])
    f = jax.nn.sigmoid(g[:, 1 * H:2 * H])
    gg = jnp.tanh(g[:, 2 * H:3 * H])
    o = jax.nn.sigmoid(g[:, 3 * H:4 * H])
    c2 = f * c + i * gg
    h2 = o * jnp.tanh(c2)
    return h2, c2


def _fused_layer(runs):
    init = tuple(
        (jnp.zeros((_B, r["H"]), jnp.float32),
         jnp.zeros((_B, r["H"]), jnp.float32))
        for r in runs
    )

    def body(t, carry):
        out = []
        for r, (h, c) in zip(runs, carry):
            te = _T - 1 - t if r["rev"] else t
            g = r["gx"](te) + jnp.dot(h, r["WhhT"],
                                      preferred_element_type=jnp.float32)
            h2, c2 = _cell(g, c, r["H"])
            if r["write"] is not None:
                r["write"](te, h2)
            out.append((h2, c2))
        return tuple(out)

    return jax.lax.fori_loop(0, _T, body, init, unroll=4)


def _mk_read(buf, off, C):
    def rd(t):
        return buf[pl.ds(t, 1)].reshape(_B, 128)[:, off:off + C]
    return rd


def _mk_write(buf, H, off):
    def wr(t, h):
        buf[pl.ds(t, 1), :, off:off + H] = h.reshape(1, _B, H)
    return wr


def _std_run(rd, WihT, b, WhhT, H, write, rev):
    def gx(t):
        return jnp.dot(rd(t), WihT, preferred_element_type=jnp.float32) + b
    return dict(gx=gx, WhhT=WhhT, H=H, write=write, rev=rev)


def _body(x_ref, eps_ref, *refs):
    nw = 87
    w = [r[...] for r in refs[:nw]]
    d_ref, mu_ref, lv_ref = refs[nw:nw + 3]
    bufA, bufB, bufC = refs[nw + 3:nw + 6]

    cur = [0]

    def take3():
        i = cur[0]
        cur[0] = i + 3
        return w[i], w[i + 1], w[i + 2]

    def gx_l0(WihT, b):
        def gx(t):
            xt = x_ref[pl.ds(t, 1), :]
            og = jax.lax.dot_general(
                xt, WihT, (((0,), (0,)), ((), ())),
                preferred_element_type=jnp.float32)
            return og + b
        return gx

    for l in range(4):
        out_off = 64 * (l % 2)
        in_off = 64 * ((l + 1) % 2)
        runs = []
        for d in range(2):
            WihT, WhhT, b = take3()
            wr = _mk_write(bufA, 32, out_off + 32 * d)
            if l == 0:
                runs.append(dict(gx=gx_l0(WihT, b), WhhT=WhhT, H=32,
                                 write=wr, rev=(d == 1)))
            else:
                runs.append(_std_run(_mk_read(bufA, in_off, 64), WihT, b,
                                     WhhT, 32, wr, d == 1))
        _fused_layer(runs)

    mean_w = [take3() for _ in range(8)]
    var_w = [take3() for _ in range(8)]
    path_io = [((bufA, 64, 64), bufB), ((bufB, 0, 128), bufC),
               ((bufC, 0, 128), bufB), ((bufB, 0, 128), None)]
    var_io = [((bufA, 64, 64), bufC), ((bufC, 0, 128), bufB),
              ((bufB, 0, 128), bufC), ((bufC, 0, 128), None)]
    for ws_all, io_all, out_ref in ((mean_w, path_io, mu_ref),
                                    (var_w, var_io, lv_ref)):
        for l in range(4):
            (src, soff, C), out = io_all[l]
            runs = []
            for d in range(2):
                WihT, WhhT, b = ws_all[2 * l + d]
                wr = None if out is None else _mk_write(out, 64, 64 * d)
                runs.append(_std_run(_mk_read(src, soff, C), WihT, b,
                                     WhhT, 64, wr, d == 1))
            finals = _fused_layer(runs)
            out_ref[:, l * 128:l * 128 + 64] = finals[0][0]
            out_ref[:, l * 128 + 64:l * 128 + 128] = finals[1][0]

    z = mu_ref[...] + jnp.exp(0.5 * lv_ref[...]) * eps_ref[...]

    runs = []
    for d in range(2):
        WihT, WhhT, b = take3()
        gz = jnp.dot(z, WihT, preferred_element_type=jnp.float32) + b
        runs.append(dict(gx=(lambda g: (lambda t: g))(gz), WhhT=WhhT, H=32,
                         write=_mk_write(bufA, 32, 32 * d), rev=(d == 1)))
    _fused_layer(runs)

    runs = []
    for d in range(2):
        WihT, WhhT, b = take3()
        runs.append(_std_run(_mk_read(bufA, 0, 64), WihT, b, WhhT, 64,
                             _mk_write(bufB, 64, 64 * d), d == 1))
    _fused_layer(runs)

    Wih3, whh3, b3 = take3()

    def body3(t, carry):
        h, c = carry
        xt = bufB[pl.ds(t, 1)].reshape(_B, 128)
        g = jax.lax.dot_general(
            Wih3, xt, (((1,), (1,)), ((), ())),
            preferred_element_type=jnp.float32)
        g = g + whh3 * h + b3
        i = jax.nn.sigmoid(g[0:1, :])
        f = jax.nn.sigmoid(g[1:2, :])
        gg = jnp.tanh(g[2:3, :])
        o = jax.nn.sigmoid(g[3:4, :])
        c2 = f * c + i * gg
        h2 = o * jnp.tanh(c2)
        d_ref[pl.ds(t, 1), :] = h2
        return (h2, c2)

    jax.lax.fori_loop(0, _T, body3,
                      (jnp.zeros((1, _B), jnp.float32),
                       jnp.zeros((1, _B), jnp.float32)), unroll=4)


def kernel(x, eps, enc_rnn1, enc_mean, enc_var, dec_rnn1, dec_rnn2, dec_rnn3):
    x2 = jnp.transpose(x[:, :, 0])

    weights = []
    for module in (enc_rnn1, enc_mean, enc_var, dec_rnn1, dec_rnn2):
        for layer in module:
            for p in layer:
                weights.extend(_prep_dir(p))
    Wih3, Whh3, bih3, bhh3 = dec_rnn3[0][0]
    weights.extend([Wih3, Whh3, (bih3 + bhh3)[:, None]])

    out_shape = [
        jax.ShapeDtypeStruct((_T, _B), jnp.float32),
        jax.ShapeDtypeStruct((_B, _FLAT), jnp.float32),
        jax.ShapeDtypeStruct((_B, _FLAT), jnp.float32),
    ]

    d_tb, mu, log_var = pl.pallas_call(
        _body,
        out_shape=out_shape,
        scratch_shapes=[pltpu.VMEM((_T, _B, 128), jnp.float32)
                        for _ in range(3)],
        compiler_params=pltpu.CompilerParams(
            vmem_limit_bytes=100 * 1024 * 1024,
        ),
        name="recurrent_autoencoder",
    )(x2, eps, *weights)

    d = jnp.transpose(d_tb)[:, :, None]
    return d, mu, log_var


# speedup vs baseline: 10.2115x; 1.3309x over previous
import jax
import jax.numpy as jnp
from jax.experimental import pallas as pl
from jax.experimental.pallas import tpu as pltpu

_T = 512
_B = 64
_FLAT = 512


def _prep_dir(p):
    Wih, Whh, bih, bhh = p
    H = Whh.shape[1]
    WhhT = jnp.transpose(Whh)
    Wp = jnp.zeros((H, 512), jnp.float32)
    for g in range(4):
        Wp = Wp.at[:, 128 * g:128 * g + H].set(WhhT[:, H * g:H * (g + 1)])
    return jnp.transpose(Wih), Wp, (bih + bhh)[None, :]


def _padg(gxc, H):
    if H == 128:
        return gxc
    Z = jnp.zeros((_B, 128 - H), jnp.float32)
    parts = []
    for g in range(4):
        parts += [gxc[:, g * H:(g + 1) * H], Z]
    return jnp.concatenate(parts, axis=1)


def _cell_p(g, c):
    i = jax.nn.sigmoid(g[:, 0:128])
    f = jax.nn.sigmoid(g[:, 128:256])
    gg = jnp.tanh(g[:, 256:384])
    o = jax.nn.sigmoid(g[:, 384:512])
    c2 = f * c + i * gg
    h2 = o * jnp.tanh(c2)
    return h2, c2


def _fused_layer(runs):
    init = tuple(
        (jnp.zeros((_B, 128), jnp.float32),
         jnp.zeros((_B, 128), jnp.float32))
        for _ in runs
    )

    def body(t, carry):
        out = []
        for r, (h, c) in zip(runs, carry):
            te = _T - 1 - t if r["rev"] else t
            g = r["gx"](te) + jnp.dot(h[:, :r["H"]], r["WhhT"],
                                      preferred_element_type=jnp.float32)
            h2, c2 = _cell_p(g, c)
            if r["write"] is not None:
                r["write"](te, h2)
            out.append((h2, c2))
        return tuple(out)

    return jax.lax.fori_loop(0, _T, body, init, unroll=4)


def _mk_read(buf, off, C):
    def rd(t):
        return buf[pl.ds(t, 1)].reshape(_B, 128)[:, off:off + C]
    return rd


def _mk_write(buf, H, off):
    def wr(t, h):
        buf[pl.ds(t, 1), :, off:off + H] = h[:, :H].reshape(1, _B, H)
    return wr


def _std_run(rd, WihT, b, WhhT, H, write, rev):
    def gx(t):
        gc = jnp.dot(rd(t), WihT, preferred_element_type=jnp.float32) + b
        return _padg(gc, H)
    return dict(gx=gx, WhhT=WhhT, H=H, write=write, rev=rev)


def _body(x_ref, eps_ref, *refs):
    nw = 87
    w = [r[...] for r in refs[:nw]]
    d_ref, mu_ref, lv_ref = refs[nw:nw + 3]
    bufA, bufB, bufC = refs[nw + 3:nw + 6]

    cur = [0]

    def take3():
        i = cur[0]
        cur[0] = i + 3
        return w[i], w[i + 1], w[i + 2]

    def gx_l0(WihT, b):
        def gx(t):
            xt = x_ref[pl.ds(t, 1), :]
            og = jax.lax.dot_general(
                xt, WihT, (((0,), (0,)), ((), ())),
                preferred_element_type=jnp.float32)
            return _padg(og + b, 32)
        return gx

    for l in range(4):
        out_off = 64 * (l % 2)
        in_off = 64 * ((l + 1) % 2)
        runs = []
        for d in range(2):
            WihT, WhhT, b = take3()
            wr = _mk_write(bufA, 32, out_off + 32 * d)
            if l == 0:
                runs.append(dict(gx=gx_l0(WihT, b), WhhT=WhhT, H=32,
                                 write=wr, rev=(d == 1)))
            else:
                runs.append(_std_run(_mk_read(bufA, in_off, 64), WihT, b,
                                     WhhT, 32, wr, d == 1))
        _fused_layer(runs)

    mean_w = [take3() for _ in range(8)]
    var_w = [take3() for _ in range(8)]
    path_io = [((bufA, 64, 64), bufB), ((bufB, 0, 128), bufC),
               ((bufC, 0, 128), bufB), ((bufB, 0, 128), None)]
    var_io = [((bufA, 64, 64), bufC), ((bufC, 0, 128), bufB),
              ((bufB, 0, 128), bufC), ((bufC, 0, 128), None)]
    for ws_all, io_all, out_ref in ((mean_w, path_io, mu_ref),
                                    (var_w, var_io, lv_ref)):
        for l in range(4):
            (src, soff, C), out = io_all[l]
            runs = []
            for d in range(2):
                WihT, WhhT, b = ws_all[2 * l + d]
                wr = None if out is None else _mk_write(out, 64, 64 * d)
                runs.append(_std_run(_mk_read(src, soff, C), WihT, b,
                                     WhhT, 64, wr, d == 1))
            finals = _fused_layer(runs)
            out_ref[:, l * 128:l * 128 + 64] = finals[0][0][:, :64]
            out_ref[:, l * 128 + 64:l * 128 + 128] = finals[1][0][:, :64]

    z = mu_ref[...] + jnp.exp(0.5 * lv_ref[...]) * eps_ref[...]

    runs = []
    for d in range(2):
        WihT, WhhT, b = take3()
        gz = _padg(jnp.dot(z, WihT, preferred_element_type=jnp.float32) + b,
                   32)
        runs.append(dict(gx=(lambda g: (lambda t: g))(gz), WhhT=WhhT, H=32,
                         write=_mk_write(bufA, 32, 32 * d), rev=(d == 1)))
    _fused_layer(runs)

    runs = []
    for d in range(2):
        WihT, WhhT, b = take3()
        runs.append(_std_run(_mk_read(bufA, 0, 64), WihT, b, WhhT, 64,
                             _mk_write(bufB, 64, 64 * d), d == 1))
    _fused_layer(runs)

    Wih3, whh3, b3 = take3()

    def body3(t, carry):
        h, c = carry
        xt = bufB[pl.ds(t, 1)].reshape(_B, 128)
        g = jax.lax.dot_general(
            Wih3, xt, (((1,), (1,)), ((), ())),
            preferred_element_type=jnp.float32)
        g = g + whh3 * h + b3
        i = jax.nn.sigmoid(g[0:1, :])
        f = jax.nn.sigmoid(g[1:2, :])
        gg = jnp.tanh(g[2:3, :])
        o = jax.nn.sigmoid(g[3:4, :])
        c2 = f * c + i * gg
        h2 = o * jnp.tanh(c2)
        d_ref[pl.ds(t, 1), :] = h2
        return (h2, c2)

    jax.lax.fori_loop(0, _T, body3,
                      (jnp.zeros((1, _B), jnp.float32),
                       jnp.zeros((1, _B), jnp.float32)), unroll=4)


def kernel(x, eps, enc_rnn1, enc_mean, enc_var, dec_rnn1, dec_rnn2, dec_rnn3):
    x2 = jnp.transpose(x[:, :, 0])

    weights = []
    for module in (enc_rnn1, enc_mean, enc_var, dec_rnn1, dec_rnn2):
        for layer in module:
            for p in layer:
                weights.extend(_prep_dir(p))
    Wih3, Whh3, bih3, bhh3 = dec_rnn3[0][0]
    weights.extend([Wih3, Whh3, (bih3 + bhh3)[:, None]])

    out_shape = [
        jax.ShapeDtypeStruct((_T, _B), jnp.float32),
        jax.ShapeDtypeStruct((_B, _FLAT), jnp.float32),
        jax.ShapeDtypeStruct((_B, _FLAT), jnp.float32),
    ]

    d_tb, mu, log_var = pl.pallas_call(
        _body,
        out_shape=out_shape,
        scratch_shapes=[pltpu.VMEM((_T, _B, 128), jnp.float32)
                        for _ in range(3)],
        compiler_params=pltpu.CompilerParams(
            vmem_limit_bytes=100 * 1024 * 1024,
        ),
        name="recurrent_autoencoder",
    )(x2, eps, *weights)

    d = jnp.transpose(d_tb)[:, :, None]
    return d, mu, log_var


# speedup vs baseline: 10.8604x; 1.0635x over previous
import jax
import jax.numpy as jnp
from jax.experimental import pallas as pl
from jax.experimental.pallas import tpu as pltpu

_T = 512
_B = 64
_FLAT = 512


def _prep_dir(p):
    Wih, Whh, bih, bhh = p
    H = Whh.shape[1]
    WhhT = jnp.transpose(Whh)
    Wp = jnp.zeros((H, 512), jnp.float32)
    for g in range(4):
        Wp = Wp.at[:, 128 * g:128 * g + H].set(WhhT[:, H * g:H * (g + 1)])
    return jnp.transpose(Wih), Wp, (bih + bhh)[None, :]


def _padg(gxc, H):
    if H == 128:
        return gxc
    Z = jnp.zeros((_B, 128 - H), jnp.float32)
    parts = []
    for g in range(4):
        parts += [gxc[:, g * H:(g + 1) * H], Z]
    return jnp.concatenate(parts, axis=1)


def _cell_p(g, c):
    i = jax.nn.sigmoid(g[:, 0:128])
    f = jax.nn.sigmoid(g[:, 128:256])
    gg = jnp.tanh(g[:, 256:384])
    o = jax.nn.sigmoid(g[:, 384:512])
    c2 = f * c + i * gg
    h2 = o * jnp.tanh(c2)
    return h2, c2


def _fused_layer(runs):
    init = tuple(
        (jnp.zeros((_B, 128), jnp.float32),
         jnp.zeros((_B, 128), jnp.float32))
        for _ in runs
    )

    def body(t, carry):
        out = []
        for r, (h, c) in zip(runs, carry):
            te = _T - 1 - t if r["rev"] else t
            g = r["gx"](te) + jnp.dot(h[:, :r["H"]], r["WhhT"],
                                      preferred_element_type=jnp.float32)
            h2, c2 = _cell_p(g, c)
            if r["write"] is not None:
                r["write"](te, h2)
            out.append((h2, c2))
        return tuple(out)

    return jax.lax.fori_loop(0, _T, body, init, unroll=8)


def _mk_read(buf, off, C):
    def rd(t):
        return buf[pl.ds(t, 1)].reshape(_B, 128)[:, off:off + C]
    return rd


def _mk_write(buf, H, off):
    def wr(t, h):
        buf[pl.ds(t, 1), :, off:off + H] = h[:, :H].reshape(1, _B, H)
    return wr


def _std_run(rd, WihT, b, WhhT, H, write, rev):
    def gx(t):
        gc = jnp.dot(rd(t), WihT, preferred_element_type=jnp.float32) + b
        return _padg(gc, H)
    return dict(gx=gx, WhhT=WhhT, H=H, write=write, rev=rev)


def _body(x_ref, eps_ref, *refs):
    nw = 87
    w = [r[...] for r in refs[:nw]]
    d_ref, mu_ref, lv_ref = refs[nw:nw + 3]
    bufA, bufB, bufC = refs[nw + 3:nw + 6]

    cur = [0]

    def take3():
        i = cur[0]
        cur[0] = i + 3
        return w[i], w[i + 1], w[i + 2]

    def gx_l0(WihT, b):
        def gx(t):
            xt = x_ref[pl.ds(t, 1), :]
            og = jax.lax.dot_general(
                xt, WihT, (((0,), (0,)), ((), ())),
                preferred_element_type=jnp.float32)
            return _padg(og + b, 32)
        return gx

    for l in range(4):
        out_off = 64 * (l % 2)
        in_off = 64 * ((l + 1) % 2)
        runs = []
        for d in range(2):
            WihT, WhhT, b = take3()
            wr = _mk_write(bufA, 32, out_off + 32 * d)
            if l == 0:
                runs.append(dict(gx=gx_l0(WihT, b), WhhT=WhhT, H=32,
                                 write=wr, rev=(d == 1)))
            else:
                runs.append(_std_run(_mk_read(bufA, in_off, 64), WihT, b,
                                     WhhT, 32, wr, d == 1))
        _fused_layer(runs)

    mean_w = [take3() for _ in range(8)]
    var_w = [take3() for _ in range(8)]
    path_io = [((bufA, 64, 64), bufB), ((bufB, 0, 128), bufC),
               ((bufC, 0, 128), bufB), ((bufB, 0, 128), None)]
    var_io = [((bufA, 64, 64), bufC), ((bufC, 0, 128), bufB),
              ((bufB, 0, 128), bufC), ((bufC, 0, 128), None)]
    for ws_all, io_all, out_ref in ((mean_w, path_io, mu_ref),
                                    (var_w, var_io, lv_ref)):
        for l in range(4):
            (src, soff, C), out = io_all[l]
            runs = []
            for d in range(2):
                WihT, WhhT, b = ws_all[2 * l + d]
                wr = None if out is None else _mk_write(out, 64, 64 * d)
                runs.append(_std_run(_mk_read(src, soff, C), WihT, b,
                                     WhhT, 64, wr, d == 1))
            finals = _fused_layer(runs)
            out_ref[:, l * 128:l * 128 + 64] = finals[0][0][:, :64]
            out_ref[:, l * 128 + 64:l * 128 + 128] = finals[1][0][:, :64]

    z = mu_ref[...] + jnp.exp(0.5 * lv_ref[...]) * eps_ref[...]

    runs = []
    for d in range(2):
        WihT, WhhT, b = take3()
        gz = _padg(jnp.dot(z, WihT, preferred_element_type=jnp.float32) + b,
                   32)
        runs.append(dict(gx=(lambda g: (lambda t: g))(gz), WhhT=WhhT, H=32,
                         write=_mk_write(bufA, 32, 32 * d), rev=(d == 1)))
    _fused_layer(runs)

    runs = []
    for d in range(2):
        WihT, WhhT, b = take3()
        runs.append(_std_run(_mk_read(bufA, 0, 64), WihT, b, WhhT, 64,
                             _mk_write(bufB, 64, 64 * d), d == 1))
    _fused_layer(runs)

    Wih3, whh3, b3 = take3()

    def body3(t, carry):
        h, c = carry
        xt = bufB[pl.ds(t, 1)].reshape(_B, 128)
        g = jax.lax.dot_general(
            Wih3, xt, (((1,), (1,)), ((), ())),
            preferred_element_type=jnp.float32)
        g = g + whh3 * h + b3
        i = jax.nn.sigmoid(g[0:1, :])
        f = jax.nn.sigmoid(g[1:2, :])
        gg = jnp.tanh(g[2:3, :])
        o = jax.nn.sigmoid(g[3:4, :])
        c2 = f * c + i * gg
        h2 = o * jnp.tanh(c2)
        d_ref[pl.ds(t, 1), :] = h2
        return (h2, c2)

    jax.lax.fori_loop(0, _T, body3,
                      (jnp.zeros((1, _B), jnp.float32),
                       jnp.zeros((1, _B), jnp.float32)), unroll=8)


def kernel(x, eps, enc_rnn1, enc_mean, enc_var, dec_rnn1, dec_rnn2, dec_rnn3):
    x2 = jnp.transpose(x[:, :, 0])

    weights = []
    for module in (enc_rnn1, enc_mean, enc_var, dec_rnn1, dec_rnn2):
        for layer in module:
            for p in layer:
                weights.extend(_prep_dir(p))
    Wih3, Whh3, bih3, bhh3 = dec_rnn3[0][0]
    weights.extend([Wih3, Whh3, (bih3 + bhh3)[:, None]])

    out_shape = [
        jax.ShapeDtypeStruct((_T, _B), jnp.float32),
        jax.ShapeDtypeStruct((_B, _FLAT), jnp.float32),
        jax.ShapeDtypeStruct((_B, _FLAT), jnp.float32),
    ]

    d_tb, mu, log_var = pl.pallas_call(
        _body,
        out_shape=out_shape,
        scratch_shapes=[pltpu.VMEM((_T, _B, 128), jnp.float32)
                        for _ in range(3)],
        compiler_params=pltpu.CompilerParams(
            vmem_limit_bytes=100 * 1024 * 1024,
        ),
        name="recurrent_autoencoder",
    )(x2, eps, *weights)

    d = jnp.transpose(d_tb)[:, :, None]
    return d, mu, log_var


# speedup vs baseline: 11.2449x; 1.0354x over previous
import jax
import jax.numpy as jnp
from jax.experimental import pallas as pl
from jax.experimental.pallas import tpu as pltpu

_T = 512
_B = 64
_FLAT = 512


def _prep_dir(p):
    Wih, Whh, bih, bhh = p
    H = Whh.shape[1]
    WhhT = jnp.transpose(Whh)
    Wp = jnp.zeros((H, 512), jnp.float32)
    for g in range(4):
        Wp = Wp.at[:, 128 * g:128 * g + H].set(WhhT[:, H * g:H * (g + 1)])
    return jnp.transpose(Wih), Wp, (bih + bhh)[None, :]


def _padg(gxc, H):
    if H == 128:
        return gxc
    Z = jnp.zeros((_B, 128 - H), jnp.float32)
    parts = []
    for g in range(4):
        parts += [gxc[:, g * H:(g + 1) * H], Z]
    return jnp.concatenate(parts, axis=1)


def _cell_p(g, c):
    i = jax.nn.sigmoid(g[:, 0:128])
    f = jax.nn.sigmoid(g[:, 128:256])
    gg = jnp.tanh(g[:, 256:384])
    o = jax.nn.sigmoid(g[:, 384:512])
    c2 = f * c + i * gg
    h2 = o * jnp.tanh(c2)
    return h2, c2


def _fused_layer(runs):
    init = tuple(
        (jnp.zeros((_B, 128), jnp.float32),
         jnp.zeros((_B, 128), jnp.float32))
        for _ in runs
    )

    def body(t, carry):
        out = []
        for r, (h, c) in zip(runs, carry):
            te = _T - 1 - t if r["rev"] else t
            g = r["gx"](te) + jnp.dot(h[:, :r["H"]], r["WhhT"],
                                      preferred_element_type=jnp.float32)
            h2, c2 = _cell_p(g, c)
            if r["write"] is not None:
                r["write"](te, h2)
            out.append((h2, c2))
        return tuple(out)

    return jax.lax.fori_loop(0, _T, body, init, unroll=16)


def _mk_read(buf, off, C):
    def rd(t):
        return buf[pl.ds(t, 1)].reshape(_B, 128)[:, off:off + C]
    return rd


def _mk_write(buf, H, off):
    def wr(t, h):
        buf[pl.ds(t, 1), :, off:off + H] = h[:, :H].reshape(1, _B, H)
    return wr


def _std_run(rd, WihT, b, WhhT, H, write, rev):
    def gx(t):
        gc = jnp.dot(rd(t), WihT, preferred_element_type=jnp.float32) + b
        return _padg(gc, H)
    return dict(gx=gx, WhhT=WhhT, H=H, write=write, rev=rev)


def _body(x_ref, eps_ref, *refs):
    nw = 87
    w = [r[...] for r in refs[:nw]]
    d_ref, mu_ref, lv_ref = refs[nw:nw + 3]
    bufA, bufB, bufC = refs[nw + 3:nw + 6]

    cur = [0]

    def take3():
        i = cur[0]
        cur[0] = i + 3
        return w[i], w[i + 1], w[i + 2]

    def gx_l0(WihT, b):
        def gx(t):
            xt = x_ref[pl.ds(t, 1), :]
            og = jax.lax.dot_general(
                xt, WihT, (((0,), (0,)), ((), ())),
                preferred_element_type=jnp.float32)
            return _padg(og + b, 32)
        return gx

    for l in range(4):
        out_off = 64 * (l % 2)
        in_off = 64 * ((l + 1) % 2)
        runs = []
        for d in range(2):
            WihT, WhhT, b = take3()
            wr = _mk_write(bufA, 32, out_off + 32 * d)
            if l == 0:
                runs.append(dict(gx=gx_l0(WihT, b), WhhT=WhhT, H=32,
                                 write=wr, rev=(d == 1)))
            else:
                runs.append(_std_run(_mk_read(bufA, in_off, 64), WihT, b,
                                     WhhT, 32, wr, d == 1))
        _fused_layer(runs)

    mean_w = [take3() for _ in range(8)]
    var_w = [take3() for _ in range(8)]
    path_io = [((bufA, 64, 64), bufB), ((bufB, 0, 128), bufC),
               ((bufC, 0, 128), bufB), ((bufB, 0, 128), None)]
    var_io = [((bufA, 64, 64), bufC), ((bufC, 0, 128), bufB),
              ((bufB, 0, 128), bufC), ((bufC, 0, 128), None)]
    for ws_all, io_all, out_ref in ((mean_w, path_io, mu_ref),
                                    (var_w, var_io, lv_ref)):
        for l in range(4):
            (src, soff, C), out = io_all[l]
            runs = []
            for d in range(2):
                WihT, WhhT, b = ws_all[2 * l + d]
                wr = None if out is None else _mk_write(out, 64, 64 * d)
                runs.append(_std_run(_mk_read(src, soff, C), WihT, b,
                                     WhhT, 64, wr, d == 1))
            finals = _fused_layer(runs)
            out_ref[:, l * 128:l * 128 + 64] = finals[0][0][:, :64]
            out_ref[:, l * 128 + 64:l * 128 + 128] = finals[1][0][:, :64]

    z = mu_ref[...] + jnp.exp(0.5 * lv_ref[...]) * eps_ref[...]

    runs = []
    for d in range(2):
        WihT, WhhT, b = take3()
        gz = _padg(jnp.dot(z, WihT, preferred_element_type=jnp.float32) + b,
                   32)
        runs.append(dict(gx=(lambda g: (lambda t: g))(gz), WhhT=WhhT, H=32,
                         write=_mk_write(bufA, 32, 32 * d), rev=(d == 1)))
    _fused_layer(runs)

    runs = []
    for d in range(2):
        WihT, WhhT, b = take3()
        runs.append(_std_run(_mk_read(bufA, 0, 64), WihT, b, WhhT, 64,
                             _mk_write(bufB, 64, 64 * d), d == 1))
    _fused_layer(runs)

    Wih3, whh3, b3 = take3()

    def body3(t, carry):
        h, c = carry
        xt = bufB[pl.ds(t, 1)].reshape(_B, 128)
        g = jax.lax.dot_general(
            Wih3, xt, (((1,), (1,)), ((), ())),
            preferred_element_type=jnp.float32)
        g = g + whh3 * h + b3
        i = jax.nn.sigmoid(g[0:1, :])
        f = jax.nn.sigmoid(g[1:2, :])
        gg = jnp.tanh(g[2:3, :])
        o = jax.nn.sigmoid(g[3:4, :])
        c2 = f * c + i * gg
        h2 = o * jnp.tanh(c2)
        d_ref[pl.ds(t, 1), :] = h2
        return (h2, c2)

    jax.lax.fori_loop(0, _T, body3,
                      (jnp.zeros((1, _B), jnp.float32),
                       jnp.zeros((1, _B), jnp.float32)), unroll=16)


def kernel(x, eps, enc_rnn1, enc_mean, enc_var, dec_rnn1, dec_rnn2, dec_rnn3):
    x2 = jnp.transpose(x[:, :, 0])

    weights = []
    for module in (enc_rnn1, enc_mean, enc_var, dec_rnn1, dec_rnn2):
        for layer in module:
            for p in layer:
                weights.extend(_prep_dir(p))
    Wih3, Whh3, bih3, bhh3 = dec_rnn3[0][0]
    weights.extend([Wih3, Whh3, (bih3 + bhh3)[:, None]])

    out_shape = [
        jax.ShapeDtypeStruct((_T, _B), jnp.float32),
        jax.ShapeDtypeStruct((_B, _FLAT), jnp.float32),
        jax.ShapeDtypeStruct((_B, _FLAT), jnp.float32),
    ]

    d_tb, mu, log_var = pl.pallas_call(
        _body,
        out_shape=out_shape,
        scratch_shapes=[pltpu.VMEM((_T, _B, 128), jnp.float32)
                        for _ in range(3)],
        compiler_params=pltpu.CompilerParams(
            vmem_limit_bytes=100 * 1024 * 1024,
        ),
        name="recurrent_autoencoder",
    )(x2, eps, *weights)

    d = jnp.transpose(d_tb)[:, :, None]
    return d, mu, log_var
